# baseline (speedup 1.0000x reference)
import functools

import jax
import jax.numpy as jnp
from jax import lax
from jax.experimental import pallas as pl
from jax.experimental.pallas import tpu as pltpu


def _pooler_body(x_hbm, w_hbm, b_ref, o_hbm,
                 x_vmem, w_vmem, o_vmem, xsems, wsem, osems,
                 *, block_b, nc):
    i = pl.program_id(0)
    row0 = i * block_b
    ch = block_b // nc

    w_cp = pltpu.make_async_copy(w_hbm, w_vmem, wsem)
    w_cp.start()
    x_cps = []
    for c in range(nc):
        cp = pltpu.make_async_copy(
            x_hbm.at[pl.ds(row0 + c * ch, ch), 0, :],
            x_vmem.at[pl.ds(c * ch, ch), :],
            xsems.at[c])
        cp.start()
        x_cps.append(cp)
    w_cp.wait()

    o_cps = []
    for c in range(nc):
        x_cps[c].wait()
        xb = x_vmem[pl.ds(c * ch, ch), :].astype(jnp.bfloat16)
        y = lax.dot_general(
            xb,
            w_vmem[...],
            dimension_numbers=(((1,), (1,)), ((), ())),
            preferred_element_type=jnp.float32,
        )
        o_vmem[pl.ds(c * ch, ch), :] = jnp.tanh(y + b_ref[...])
        cp = pltpu.make_async_copy(
            o_vmem.at[pl.ds(c * ch, ch), :],
            o_hbm.at[pl.ds(row0 + c * ch, ch), :],
            osems.at[c])
        cp.start()
        o_cps.append(cp)
    for cp in o_cps:
        cp.wait()


def kernel(x, weight, bias, *, block_b=512, nc=4):
    B, S, D = x.shape
    assert weight.shape == (D, D) and bias.shape == (D,)
    assert B % block_b == 0 and block_b % nc == 0

    b2d = bias.reshape(1, D).astype(jnp.float32)
    grid = (B // block_b,)

    cost = pl.CostEstimate(
        flops=2 * B * D * D,
        transcendentals=B * D,
        bytes_accessed=(D * D * jnp.dtype(weight.dtype).itemsize
                        + B * D * jnp.dtype(x.dtype).itemsize
                        + D * 4
                        + B * D * jnp.dtype(x.dtype).itemsize),
    )

    return pl.pallas_call(
        functools.partial(_pooler_body, block_b=block_b, nc=nc),
        out_shape=jax.ShapeDtypeStruct((B, D), x.dtype),
        grid=grid,
        in_specs=[
            pl.BlockSpec(memory_space=pl.ANY),
            pl.BlockSpec(memory_space=pl.ANY),
            pl.BlockSpec((1, D), lambda b: (0, 0)),
        ],
        out_specs=pl.BlockSpec(memory_space=pl.ANY),
        scratch_shapes=[
            pltpu.VMEM((block_b, D), jnp.float32),
            pltpu.VMEM((D, D), jnp.bfloat16),
            pltpu.VMEM((block_b, D), jnp.float32),
            pltpu.SemaphoreType.DMA((nc,)),
            pltpu.SemaphoreType.DMA,
            pltpu.SemaphoreType.DMA((nc,)),
        ],
        compiler_params=pltpu.CompilerParams(
            dimension_semantics=("parallel",),
            vmem_limit_bytes=48 * 1024 * 1024,
        ),
        cost_estimate=cost,
    )(x, weight, b2d)


# speedup vs baseline: 1.3618x; 1.3618x over previous
import functools

import jax
import jax.numpy as jnp
from jax import lax
from jax.experimental import pallas as pl
from jax.experimental.pallas import tpu as pltpu


def _pooler_body(x_hbm, w_ref, b_ref, o_ref, x_vmem, sems, *, block_b, nsplit):
    i = pl.program_id(0)
    ch = block_b // nsplit
    cps = []
    for c in range(nsplit):
        cp = pltpu.make_async_copy(
            x_hbm.at[pl.ds(i * block_b + c * ch, ch), 0, :],
            x_vmem.at[pl.ds(c * ch, ch), :],
            sems.at[c])
        cp.start()
        cps.append(cp)
    for cp in cps:
        cp.wait()
    xb = x_vmem[...].astype(jnp.bfloat16)
    y = lax.dot_general(
        xb,
        w_ref[...],
        dimension_numbers=(((1,), (1,)), ((), ())),
        preferred_element_type=jnp.float32,
    )
    y = y + b_ref[...]
    o_ref[...] = jnp.tanh(y).astype(o_ref.dtype)


def kernel(x, weight, bias, *, block_b=512, nsplit=4):
    B, S, D = x.shape
    assert weight.shape == (D, D) and bias.shape == (D,)
    assert B % block_b == 0 and block_b % nsplit == 0

    b2d = bias.reshape(1, D).astype(jnp.float32)
    grid = (B // block_b,)

    cost = pl.CostEstimate(
        flops=2 * B * D * D,
        transcendentals=B * D,
        bytes_accessed=(D * D * jnp.dtype(weight.dtype).itemsize
                        + B * D * jnp.dtype(x.dtype).itemsize
                        + D * 4
                        + B * D * jnp.dtype(x.dtype).itemsize),
    )

    return pl.pallas_call(
        functools.partial(_pooler_body, block_b=block_b, nsplit=nsplit),
        out_shape=jax.ShapeDtypeStruct((B, D), x.dtype),
        grid=grid,
        in_specs=[
            pl.BlockSpec(memory_space=pl.ANY),
            pl.BlockSpec((D, D), lambda b: (0, 0)),
            pl.BlockSpec((1, D), lambda b: (0, 0)),
        ],
        out_specs=pl.BlockSpec((block_b, D), lambda b: (b, 0)),
        scratch_shapes=[
            pltpu.VMEM((block_b, D), jnp.float32),
            pltpu.SemaphoreType.DMA((nsplit,)),
        ],
        compiler_params=pltpu.CompilerParams(
            dimension_semantics=("parallel",),
            vmem_limit_bytes=48 * 1024 * 1024,
        ),
        cost_estimate=cost,
    )(x, weight, b2d)
